# baseline (speedup 1.0000x reference)
import functools

import jax
import jax.numpy as jnp
from jax import lax
from jax.experimental import pallas as pl
from jax.experimental.pallas import tpu as pltpu
from jax.experimental.pallas import tpu_sc as plsc

NL = 10000
NC = 10000
E = 320000
D = 128

NCORES = 2
NSUB = 16
NW = NCORES * NSUB
CHUNK = 125
CP = 128
CPW = E // (NW * CHUNK)
NBUF = 2
NIBUF = 4

SEG_PAD = 10240
DUSTBIN = SEG_PAD
ACC_ROWS = SEG_PAD + 8
ROWS_PT = SEG_PAD // NSUB


def _seg_sum_partials(vals, gidx3, sidx3, zeros_hbm):
    mesh = plsc.VectorSubcoreMesh(core_axis_name="c", subcore_axis_name="s")

    @functools.partial(
        pl.kernel,
        out_type=jax.ShapeDtypeStruct((NCORES, SEG_PAD, D), jnp.float32),
        mesh=mesh,
        scratch_types=[
            pltpu.VMEM_SHARED((ACC_ROWS, D), jnp.float32),
            pltpu.VMEM((NIBUF, CP), jnp.int32),
            pltpu.VMEM((NIBUF, CP), jnp.int32),
            pltpu.VMEM((NBUF, CP, D), jnp.float32),
            [pltpu.SemaphoreType.DMA] * NIBUF,
            [pltpu.SemaphoreType.DMA] * NIBUF,
            [pltpu.SemaphoreType.DMA] * NBUF,
            [pltpu.SemaphoreType.DMA] * NBUF,
        ],
    )
    def k(vals_hbm, gidx_hbm, sidx_hbm, z_hbm, out_hbm, acc_sh, gi, si,
          rows, gisems, sisems, gsems, ssems):
        c = lax.axis_index("c")
        s = lax.axis_index("s")
        wid = c * NSUB + s

        def start_idx(j, t):
            pltpu.async_copy(gidx_hbm.at[wid, j], gi.at[t], gisems[t])
            pltpu.async_copy(sidx_hbm.at[wid, j], si.at[t], sisems[t])

        def wait_gidx(j, t):
            pltpu.make_async_copy(gidx_hbm.at[wid, j], gi.at[t],
                                  gisems[t]).wait()

        def wait_sidx(j, t):
            pltpu.make_async_copy(sidx_hbm.at[wid, j], si.at[t],
                                  sisems[t]).wait()

        def start_gather(j, t, b):
            pltpu.async_copy(vals_hbm.at[gi.at[t]], rows.at[b], gsems[b])

        def wait_gather(j, t, b):
            pltpu.make_async_copy(vals_hbm.at[gi.at[t]], rows.at[b],
                                  gsems[b]).wait()

        def start_scatter(j, t, b):
            pltpu.async_copy(rows.at[b], acc_sh.at[si.at[t]], ssems[b],
                             add=True)

        def wait_scatter(j, t, b):
            pltpu.make_async_copy(rows.at[b], acc_sh.at[si.at[t]],
                                  ssems[b]).wait()

        for j in range(NIBUF - 1):
            start_idx(j, j)
        pltpu.sync_copy(z_hbm, acc_sh.at[pl.ds(s * ROWS_PT, ROWS_PT)])
        plsc.subcore_barrier()

        def make_iter(tj, tg, ti, do_ws, do_idx, do_gather):
            def run(j):
                b = tj % NBUF
                if do_ws:
                    wait_scatter(j - 1, (tj - 1) % NIBUF, (tj - 1) % NBUF)
                if do_idx:
                    start_idx(j + NIBUF - 1, ti)
                if do_gather:
                    wait_gidx(j + 1, tg)
                    start_gather(j + 1, tg, (tj + 1) % NBUF)
                wait_gather(j, tj, b)
                wait_sidx(j, tj)
                start_scatter(j, tj, b)
            return run

        wait_gidx(0, 0)
        start_gather(0, 0, 0)
        make_iter(0, 1, 3 % NIBUF, False, True, True)(0)
        def round_body(r, carry):
            j0 = 1 + r * NIBUF
            for u in range(NIBUF):
                tj = (1 + u) % NIBUF
                make_iter(tj, (tj + 1) % NIBUF, (tj + NIBUF - 1) % NIBUF,
                          True, True, True)(j0 + u)
            return carry

        n_steady = (CPW - NIBUF) // NIBUF
        lax.fori_loop(0, n_steady, round_body, 0, unroll=False)
        for u in range(NIBUF - 1):
            j = CPW - NIBUF + 1 + u
            tj = j % NIBUF
            make_iter(tj, (tj + 1) % NIBUF, 0, True, False,
                      u < NIBUF - 2)(j)
        wait_scatter(CPW - 1, (CPW - 1) % NIBUF, (CPW - 1) % NBUF)

        plsc.subcore_barrier()

        pltpu.sync_copy(acc_sh.at[pl.ds(s * ROWS_PT, ROWS_PT)],
                        out_hbm.at[c, pl.ds(s * ROWS_PT, ROWS_PT)])

    return k(vals, gidx3, sidx3, zeros_hbm)


BR = 2000


def _dense_in(x, W, b):
    n, kdim = x.shape

    def body(x_ref, w_ref, b_ref, o_ref):
        o_ref[...] = (
            jnp.dot(x_ref[...], w_ref[...], preferred_element_type=jnp.float32)
            + b_ref[...])

    return pl.pallas_call(
        body,
        grid=(n // BR,),
        in_specs=[
            pl.BlockSpec((BR, kdim), lambda i: (i, 0)),
            pl.BlockSpec((kdim, D), lambda i: (0, 0)),
            pl.BlockSpec((1, D), lambda i: (0, 0)),
        ],
        out_specs=pl.BlockSpec((BR, D), lambda i: (i, 0)),
        out_shape=jax.ShapeDtypeStruct((n, D), jnp.float32),
    )(x, W, b)


def _dense_mid(cpart, W, b):
    def body(cp_ref, w_ref, b_ref, ce_ref, m2_ref):
        ce = jnp.maximum(cp_ref[0] + cp_ref[1], 0.0)
        ce_ref[...] = ce
        m2_ref[...] = (
            jnp.dot(ce, w_ref[...], preferred_element_type=jnp.float32)
            + b_ref[...])

    return pl.pallas_call(
        body,
        grid=(NC // BR,),
        in_specs=[
            pl.BlockSpec((2, BR, D), lambda i: (0, i, 0)),
            pl.BlockSpec((D, D), lambda i: (0, 0)),
            pl.BlockSpec((1, D), lambda i: (0, 0)),
        ],
        out_specs=[
            pl.BlockSpec((BR, D), lambda i: (i, 0)),
            pl.BlockSpec((BR, D), lambda i: (i, 0)),
        ],
        out_shape=[
            jax.ShapeDtypeStruct((NC, D), jnp.float32),
            jax.ShapeDtypeStruct((NC, D), jnp.float32),
        ],
    )(cpart, W, b)


def _dense_out(lpart, g, b):
    def body(lp_ref, g_ref, b_ref, o_ref):
        x = jnp.maximum(lp_ref[0] + lp_ref[1], 0.0)
        mu = jnp.mean(x, axis=-1, keepdims=True)
        var = jnp.mean((x - mu) ** 2, axis=-1, keepdims=True)
        o_ref[...] = (x - mu) * lax.rsqrt(var + 1e-5) * g_ref[...] + b_ref[...]

    return pl.pallas_call(
        body,
        grid=(NL // BR,),
        in_specs=[
            pl.BlockSpec((2, BR, D), lambda i: (0, i, 0)),
            pl.BlockSpec((1, D), lambda i: (0, 0)),
            pl.BlockSpec((1, D), lambda i: (0, 0)),
        ],
        out_specs=pl.BlockSpec((BR, D), lambda i: (i, 0)),
        out_shape=jax.ShapeDtypeStruct((NL, D), jnp.float32),
    )(lpart, g, b)


def _tie(pre):
    z = pre.reshape(-1, 2, D)
    rc = z[:, ::-1, :].reshape(-1, D)
    return jnp.concatenate([pre, rc], axis=1)


def kernel(vlabels, edge_index, Wl2c0, bl2c0, Wc2l0, bc2l0, lng0, lnb0,
           Wl2c1, bl2c1, Wc2l1, bc2l1, lng1, lnb1,
           Wl2c2, bl2c2, Wc2l2, bc2l2, lng2, lnb2):
    src3 = edge_index[0].reshape(NW, CPW, CHUNK)
    dst3 = edge_index[1].reshape(NW, CPW, CHUNK)
    gpad = jnp.zeros((NW, CPW, CP - CHUNK), jnp.int32)
    spad = jnp.full((NW, CPW, CP - CHUNK), DUSTBIN, jnp.int32)
    src_g = jnp.concatenate([src3, gpad], axis=-1)
    src_s = jnp.concatenate([src3, spad], axis=-1)
    dst_g = jnp.concatenate([dst3, gpad], axis=-1)
    dst_s = jnp.concatenate([dst3, spad], axis=-1)
    zeros_hbm = jnp.zeros((ROWS_PT, D), jnp.float32)
    params = [
        (Wl2c0, bl2c0, Wc2l0, bc2l0, lng0, lnb0),
        (Wl2c1, bl2c1, Wc2l1, bc2l1, lng1, lnb1),
        (Wl2c2, bl2c2, Wc2l2, bc2l2, lng2, lnb2),
    ]
    h = vlabels
    cembs = None
    for (Wa, ba, Wb, bb, g, b) in params:
        m = _dense_in(h, Wa, ba.reshape(1, D))
        cpart = _seg_sum_partials(m, src_g, dst_s, zeros_hbm)
        cembs, m2 = _dense_mid(cpart, Wb, bb.reshape(1, D))
        lpart = _seg_sum_partials(m2, dst_g, src_s, zeros_hbm)
        pre = _dense_out(lpart, g.reshape(1, D), b.reshape(1, D))
        h = _tie(pre)
    return (h, cembs)


# speedup vs baseline: 1.0557x; 1.0557x over previous
import functools

import jax
import jax.numpy as jnp
from jax import lax
from jax.experimental import pallas as pl
from jax.experimental.pallas import tpu as pltpu
from jax.experimental.pallas import tpu_sc as plsc

NL = 10000
NC = 10000
E = 320000
D = 128

NCORES = 2
NSUB = 16
NW = NCORES * NSUB
CHUNK = 125
CP = 128
CPW = E // (NW * CHUNK)
NBUF = 2
NIBUF = 4

SEG_PAD = 10240
DUSTBIN = SEG_PAD
ACC_ROWS = SEG_PAD + 8
ROWS_PT = SEG_PAD // NSUB


def _seg_sum_partials(vals, gidx3, zeros_hbm):
    mesh = plsc.VectorSubcoreMesh(core_axis_name="c", subcore_axis_name="s")

    @functools.partial(
        pl.kernel,
        out_type=jax.ShapeDtypeStruct((NCORES, SEG_PAD, D), jnp.float32),
        mesh=mesh,
        scratch_types=[
            pltpu.VMEM_SHARED((ACC_ROWS, D), jnp.float32),
            pltpu.VMEM((2, CP), jnp.int32),
            pltpu.VMEM((CP, D), jnp.float32),
            pltpu.SemaphoreType.DMA,
        ],
    )
    def k(vals_hbm, idx_hbm, z_hbm, out_hbm, acc_sh, ix, rows, sem):
        c = lax.axis_index("c")
        s = lax.axis_index("s")
        wid = c * NSUB + s

        pltpu.sync_copy(z_hbm, acc_sh.at[pl.ds(s * ROWS_PT, ROWS_PT)])
        plsc.subcore_barrier()

        def body(j, carry):
            pltpu.sync_copy(idx_hbm.at[wid, j], ix)
            pltpu.async_copy(vals_hbm.at[ix.at[0]], rows, sem).wait()
            pltpu.sync_copy(rows, acc_sh.at[ix.at[1]], add=True)
            return carry

        lax.fori_loop(0, CPW, body, 0)
        plsc.subcore_barrier()

        pltpu.sync_copy(acc_sh.at[pl.ds(s * ROWS_PT, ROWS_PT)],
                        out_hbm.at[c, pl.ds(s * ROWS_PT, ROWS_PT)])

    return k(vals, gidx3, zeros_hbm)


BR = 2000


def _dense_in(x, W, b):
    n, kdim = x.shape

    def body(x_ref, w_ref, b_ref, o_ref):
        o_ref[...] = (
            jnp.dot(x_ref[...], w_ref[...], preferred_element_type=jnp.float32)
            + b_ref[...])

    return pl.pallas_call(
        body,
        grid=(n // BR,),
        in_specs=[
            pl.BlockSpec((BR, kdim), lambda i: (i, 0)),
            pl.BlockSpec((kdim, D), lambda i: (0, 0)),
            pl.BlockSpec((1, D), lambda i: (0, 0)),
        ],
        out_specs=pl.BlockSpec((BR, D), lambda i: (i, 0)),
        out_shape=jax.ShapeDtypeStruct((n, D), jnp.float32),
    )(x, W, b)


def _dense_mid(cpart, W, b):
    def body(cp_ref, w_ref, b_ref, ce_ref, m2_ref):
        ce = jnp.maximum(cp_ref[0] + cp_ref[1], 0.0)
        ce_ref[...] = ce
        m2_ref[...] = (
            jnp.dot(ce, w_ref[...], preferred_element_type=jnp.float32)
            + b_ref[...])

    return pl.pallas_call(
        body,
        grid=(NC // BR,),
        in_specs=[
            pl.BlockSpec((2, BR, D), lambda i: (0, i, 0)),
            pl.BlockSpec((D, D), lambda i: (0, 0)),
            pl.BlockSpec((1, D), lambda i: (0, 0)),
        ],
        out_specs=[
            pl.BlockSpec((BR, D), lambda i: (i, 0)),
            pl.BlockSpec((BR, D), lambda i: (i, 0)),
        ],
        out_shape=[
            jax.ShapeDtypeStruct((NC, D), jnp.float32),
            jax.ShapeDtypeStruct((NC, D), jnp.float32),
        ],
    )(cpart, W, b)


def _ln_tie(lp_ref, g_ref, b_ref):
    x = jnp.maximum(lp_ref[0] + lp_ref[1], 0.0)
    mu = jnp.mean(x, axis=-1, keepdims=True)
    var = jnp.mean((x - mu) ** 2, axis=-1, keepdims=True)
    pre = (x - mu) * lax.rsqrt(var + 1e-5) * g_ref[...] + b_ref[...]
    up = pltpu.roll(pre, BR - 1, 0)
    dn = pltpu.roll(pre, 1, 0)
    par = lax.broadcasted_iota(jnp.int32, (BR, D), 0) % 2
    rc = jnp.where(par == 0, up, dn)
    return pre, rc


def _fused_out_in(lpart, g, b, Wa, ba):
    def body(lp_ref, g_ref, b_ref, w1_ref, w2_ref, ba_ref, o_ref):
        pre, rc = _ln_tie(lp_ref, g_ref, b_ref)
        o_ref[...] = (
            jnp.dot(pre, w1_ref[...], preferred_element_type=jnp.float32)
            + jnp.dot(rc, w2_ref[...], preferred_element_type=jnp.float32)
            + ba_ref[...])

    return pl.pallas_call(
        body,
        grid=(NL // BR,),
        in_specs=[
            pl.BlockSpec((2, BR, D), lambda i: (0, i, 0)),
            pl.BlockSpec((1, D), lambda i: (0, 0)),
            pl.BlockSpec((1, D), lambda i: (0, 0)),
            pl.BlockSpec((D, D), lambda i: (0, 0)),
            pl.BlockSpec((D, D), lambda i: (0, 0)),
            pl.BlockSpec((1, D), lambda i: (0, 0)),
        ],
        out_specs=pl.BlockSpec((BR, D), lambda i: (i, 0)),
        out_shape=jax.ShapeDtypeStruct((NL, D), jnp.float32),
    )(lpart, g, b, Wa[:D], Wa[D:], ba)


def _final_tie(lpart, g, b):
    def body(lp_ref, g_ref, b_ref, o_ref):
        pre, rc = _ln_tie(lp_ref, g_ref, b_ref)
        o_ref[:, :D] = pre
        o_ref[:, D:] = rc

    return pl.pallas_call(
        body,
        grid=(NL // BR,),
        in_specs=[
            pl.BlockSpec((2, BR, D), lambda i: (0, i, 0)),
            pl.BlockSpec((1, D), lambda i: (0, 0)),
            pl.BlockSpec((1, D), lambda i: (0, 0)),
        ],
        out_specs=pl.BlockSpec((BR, 2 * D), lambda i: (i, 0)),
        out_shape=jax.ShapeDtypeStruct((NL, 2 * D), jnp.float32),
    )(lpart, g, b)


def kernel(vlabels, edge_index, Wl2c0, bl2c0, Wc2l0, bc2l0, lng0, lnb0,
           Wl2c1, bl2c1, Wc2l1, bc2l1, lng1, lnb1,
           Wl2c2, bl2c2, Wc2l2, bc2l2, lng2, lnb2):
    src3 = edge_index[0].reshape(NW, CPW, CHUNK)
    dst3 = edge_index[1].reshape(NW, CPW, CHUNK)
    gpad = jnp.zeros((NW, CPW, CP - CHUNK), jnp.int32)
    spad = DUSTBIN + jnp.broadcast_to(
        (jnp.arange(NW, dtype=jnp.int32) % 8)[:, None, None],
        (NW, CPW, CP - CHUNK))
    src_g = jnp.concatenate([src3, gpad], axis=-1)
    src_s = jnp.concatenate([src3, spad], axis=-1)
    dst_g = jnp.concatenate([dst3, gpad], axis=-1)
    dst_s = jnp.concatenate([dst3, spad], axis=-1)
    idx_l2c = jnp.stack([src_g, dst_s], axis=2)
    idx_c2l = jnp.stack([dst_g, src_s], axis=2)
    zeros_hbm = jnp.zeros((ROWS_PT, D), jnp.float32)
    params = [
        (Wl2c0, bl2c0, Wc2l0, bc2l0, lng0, lnb0),
        (Wl2c1, bl2c1, Wc2l1, bc2l1, lng1, lnb1),
        (Wl2c2, bl2c2, Wc2l2, bc2l2, lng2, lnb2),
    ]
    cembs = None
    lpart = None
    for i, (Wa, ba, Wb, bb, g, b) in enumerate(params):
        if i == 0:
            m = _dense_in(vlabels, Wa, ba.reshape(1, D))
        else:
            pg, pb = params[i - 1][4], params[i - 1][5]
            m = _fused_out_in(lpart, pg.reshape(1, D), pb.reshape(1, D),
                              Wa, ba.reshape(1, D))
        cpart = _seg_sum_partials(m, idx_l2c, zeros_hbm)
        cembs, m2 = _dense_mid(cpart, Wb, bb.reshape(1, D))
        lpart = _seg_sum_partials(m2, idx_c2l, zeros_hbm)
    h = _final_tie(lpart, lng2.reshape(1, D), lnb2.reshape(1, D))
    return (h, cembs)
